# baseline (speedup 1.0000x reference)
import functools

import jax
import jax.numpy as jnp
from jax import lax
from jax.experimental import pallas as pl
from jax.experimental.pallas import tpu as pltpu
from jax.experimental.pallas import tpu_sc as plsc

N = 10000
E = 320000
D = 128
H = D // 2

CHUNK = 256
CHUNKS = 80
EPT = CHUNKS * CHUNK
EPAD = 16 * EPT
NPAD = 10240
ROWS_PER_TILE = NPAD // 16
WB = 128
WB_ITERS = ROWS_PER_TILE // WB


def _silu(v):
    return v * (1.0 / (1.0 + jnp.exp(-v)))


def _ln(v, eps=1e-6):
    m = jnp.mean(v, axis=-1, keepdims=True)
    var = jnp.mean((v - m) * (v - m), axis=-1, keepdims=True)
    return (v - m) / jnp.sqrt(var + eps)


def _split_halves(h):
    return jnp.moveaxis(h.reshape(h.shape[0], 2, H), 1, 0)


_BLK = 2000


def _pre_body(x_ref, g_ref, b_ref, o_ref):
    h = _ln(x_ref[...]) * g_ref[...] + b_ref[...]
    o_ref[...] = _split_halves(_silu(h))


def _tc_pre(x, gamma, beta):
    out = pl.pallas_call(
        _pre_body,
        grid=(N // _BLK,),
        in_specs=[
            pl.BlockSpec((_BLK, D), lambda i: (i, 0)),
            pl.BlockSpec((1, D), lambda i: (0, 0)),
            pl.BlockSpec((1, D), lambda i: (0, 0)),
        ],
        out_specs=pl.BlockSpec((2, _BLK, H), lambda i: (0, i, 0)),
        out_shape=jax.ShapeDtypeStruct((2, N, H), jnp.float32),
    )(x, gamma.reshape(1, D), beta.reshape(1, D))
    return out.reshape(2 * N, H)


def _mid_body(p_ref, w_ref, b_ref, o_ref):
    agg = jnp.concatenate([p_ref[0], p_ref[1]], axis=1)
    t = lax.dot_general(agg, w_ref[...], (((1,), (0,)), ((), ())),
                        preferred_element_type=jnp.float32) + b_ref[...]
    o_ref[...] = _split_halves(_silu(_ln(t)))


def _tc_mid(p, w, b):
    out = pl.pallas_call(
        _mid_body,
        grid=(N // _BLK,),
        in_specs=[
            pl.BlockSpec((2, _BLK, H), lambda i: (0, i, 0)),
            pl.BlockSpec((D, D), lambda i: (0, 0)),
            pl.BlockSpec((1, D), lambda i: (0, 0)),
        ],
        out_specs=pl.BlockSpec((2, _BLK, H), lambda i: (0, i, 0)),
        out_shape=jax.ShapeDtypeStruct((2, N, H), jnp.float32),
    )(p, w, b.reshape(1, D))
    return out.reshape(2 * N, H)


def _fin_body(p_ref, w_ref, b_ref, x_ref, o_ref):
    agg = jnp.concatenate([p_ref[0], p_ref[1]], axis=1)
    t = lax.dot_general(agg, w_ref[...], (((1,), (0,)), ((), ())),
                        preferred_element_type=jnp.float32) + b_ref[...]
    o_ref[...] = t + x_ref[...]


def _tc_fin(p, w, b, x):
    return pl.pallas_call(
        _fin_body,
        grid=(N // _BLK,),
        in_specs=[
            pl.BlockSpec((2, _BLK, H), lambda i: (0, i, 0)),
            pl.BlockSpec((D, D), lambda i: (0, 0)),
            pl.BlockSpec((1, D), lambda i: (0, 0)),
            pl.BlockSpec((_BLK, D), lambda i: (i, 0)),
        ],
        out_specs=pl.BlockSpec((_BLK, D), lambda i: (i, 0)),
        out_shape=jax.ShapeDtypeStruct((N, D), jnp.float32),
    )(p, w, b.reshape(1, D), x)


@functools.partial(
    pl.kernel,
    mesh=plsc.VectorSubcoreMesh(core_axis_name="c", subcore_axis_name="s"),
    out_type=jax.ShapeDtypeStruct((2, NPAD, H), jnp.float32),
    compiler_params=pltpu.CompilerParams(use_tc_tiling_on_sc=False),
    scratch_types=[
        pltpu.VMEM((CHUNKS, CHUNK), jnp.int32),
        pltpu.VMEM((CHUNKS, CHUNK), jnp.int32),
        pltpu.VMEM((CHUNK, H), jnp.float32),
        pltpu.VMEM((CHUNK, H), jnp.float32),
        pltpu.VMEM((WB, H), jnp.float32),
        pltpu.VMEM_SHARED((NPAD, H), jnp.float32),
        pltpu.SemaphoreType.DMA,
        pltpu.SemaphoreType.DMA,
    ],
)
def _sc_agg(h_hbm, src_hbm, dst_hbm, zeros_hbm, out_hbm,
            src_v, dst_v, buf0, buf1, zbuf, agg_sh, sem0, sem1):
    c = lax.axis_index("c")
    s = lax.axis_index("s")

    pltpu.sync_copy(src_hbm.at[c, s], src_v)
    pltpu.sync_copy(dst_hbm.at[s], dst_v)

    pltpu.sync_copy(zeros_hbm, zbuf)
    base = s * ROWS_PER_TILE

    def _zero(k, _):
        pltpu.sync_copy(zbuf, agg_sh.at[pl.ds(base + k * WB, WB)])
        return 0

    lax.fori_loop(0, WB_ITERS, _zero, 0)
    plsc.subcore_barrier()

    pltpu.async_copy(h_hbm.at[src_v.at[0]], buf0, sem0)

    def _step(i, _):
        j = i * 2
        pltpu.async_copy(h_hbm.at[src_v.at[j + 1]], buf1, sem1)
        pltpu.make_async_copy(h_hbm.at[src_v.at[j]], buf0, sem0).wait()
        pltpu.sync_copy(buf0, agg_sh.at[dst_v.at[j]], add=True)
        pltpu.async_copy(h_hbm.at[src_v.at[j + 2]], buf0, sem0)
        pltpu.make_async_copy(h_hbm.at[src_v.at[j + 1]], buf1, sem1).wait()
        pltpu.sync_copy(buf1, agg_sh.at[dst_v.at[j + 1]], add=True)
        return 0

    lax.fori_loop(0, (CHUNKS - 2) // 2, _step, 0)
    pltpu.async_copy(h_hbm.at[src_v.at[CHUNKS - 1]], buf1, sem1)
    pltpu.make_async_copy(h_hbm.at[src_v.at[CHUNKS - 2]], buf0, sem0).wait()
    pltpu.sync_copy(buf0, agg_sh.at[dst_v.at[CHUNKS - 2]], add=True)
    pltpu.make_async_copy(h_hbm.at[src_v.at[CHUNKS - 1]], buf1, sem1).wait()
    pltpu.sync_copy(buf1, agg_sh.at[dst_v.at[CHUNKS - 1]], add=True)

    plsc.subcore_barrier()

    def _wb(k, _):
        pltpu.sync_copy(agg_sh.at[pl.ds(base + k * WB, WB)], zbuf)
        pltpu.sync_copy(zbuf, out_hbm.at[c, pl.ds(base + k * WB, WB)])
        return 0

    lax.fori_loop(0, WB_ITERS, _wb, 0)


def kernel(x, edge_index, ln_gamma, ln_beta, W1, b1, W2, b2):
    src = edge_index[0]
    dst = edge_index[1]

    pad = EPAD - E
    src_p = jnp.concatenate([src, jnp.zeros((pad,), jnp.int32)])
    dst_p = jnp.concatenate([dst, jnp.full((pad,), NPAD - 1, jnp.int32)])
    src3 = src_p.reshape(16, CHUNKS, CHUNK)
    src4 = jnp.stack([src3, src3 + N])
    dst3 = dst_p.reshape(16, CHUNKS, CHUNK)
    zeros_tile = jnp.zeros((WB, H), jnp.float32)

    h1 = _tc_pre(x, ln_gamma, ln_beta)
    p1 = _sc_agg(h1, src4, dst3, zeros_tile)
    h2 = _tc_mid(p1, W1, b1)
    p2 = _sc_agg(h2, src4, dst3, zeros_tile)
    return _tc_fin(p2, W2, b2, x)


# speedup vs baseline: 1.4058x; 1.4058x over previous
import functools

import jax
import jax.numpy as jnp
from jax import lax
from jax.experimental import pallas as pl
from jax.experimental.pallas import tpu as pltpu
from jax.experimental.pallas import tpu_sc as plsc

N = 10000
E = 320000
D = 128
H = D // 2

CHUNK = 128
CHUNKS = 158
EPT = CHUNKS * CHUNK
EPAD = 16 * EPT
NPAD = 10240
ROWS_PER_TILE = NPAD // 16
WB = 128
WB_ITERS = ROWS_PER_TILE // WB


def _silu(v):
    return v * (1.0 / (1.0 + jnp.exp(-v)))


def _ln(v, eps=1e-6):
    m = jnp.mean(v, axis=-1, keepdims=True)
    var = jnp.mean((v - m) * (v - m), axis=-1, keepdims=True)
    return (v - m) / jnp.sqrt(var + eps)


def _split_halves(h):
    return jnp.moveaxis(h.reshape(h.shape[0], 2, H), 1, 0)


_BLK = 2000


def _pre_body(x_ref, g_ref, b_ref, o_ref):
    h = _ln(x_ref[...]) * g_ref[...] + b_ref[...]
    o_ref[...] = _split_halves(_silu(h))


def _tc_pre(x, gamma, beta):
    out = pl.pallas_call(
        _pre_body,
        grid=(N // _BLK,),
        in_specs=[
            pl.BlockSpec((_BLK, D), lambda i: (i, 0)),
            pl.BlockSpec((1, D), lambda i: (0, 0)),
            pl.BlockSpec((1, D), lambda i: (0, 0)),
        ],
        out_specs=pl.BlockSpec((2, _BLK, H), lambda i: (0, i, 0)),
        out_shape=jax.ShapeDtypeStruct((2, N, H), jnp.float32),
    )(x, gamma.reshape(1, D), beta.reshape(1, D))
    return out.reshape(2 * N, H)


def _mid_body(p_ref, w_ref, b_ref, o_ref):
    agg = jnp.concatenate([p_ref[0], p_ref[1]], axis=1)
    t = lax.dot_general(agg, w_ref[...], (((1,), (0,)), ((), ())),
                        preferred_element_type=jnp.float32) + b_ref[...]
    o_ref[...] = _split_halves(_silu(_ln(t)))


def _tc_mid(p, w, b):
    out = pl.pallas_call(
        _mid_body,
        grid=(N // _BLK,),
        in_specs=[
            pl.BlockSpec((2, _BLK, H), lambda i: (0, i, 0)),
            pl.BlockSpec((D, D), lambda i: (0, 0)),
            pl.BlockSpec((1, D), lambda i: (0, 0)),
        ],
        out_specs=pl.BlockSpec((2, _BLK, H), lambda i: (0, i, 0)),
        out_shape=jax.ShapeDtypeStruct((2, N, H), jnp.float32),
    )(p, w, b.reshape(1, D))
    return out.reshape(2 * N, H)


def _fin_body(p_ref, w_ref, b_ref, x_ref, o_ref):
    agg = jnp.concatenate([p_ref[0], p_ref[1]], axis=1)
    t = lax.dot_general(agg, w_ref[...], (((1,), (0,)), ((), ())),
                        preferred_element_type=jnp.float32) + b_ref[...]
    o_ref[...] = t + x_ref[...]


def _tc_fin(p, w, b, x):
    return pl.pallas_call(
        _fin_body,
        grid=(N // _BLK,),
        in_specs=[
            pl.BlockSpec((2, _BLK, H), lambda i: (0, i, 0)),
            pl.BlockSpec((D, D), lambda i: (0, 0)),
            pl.BlockSpec((1, D), lambda i: (0, 0)),
            pl.BlockSpec((_BLK, D), lambda i: (i, 0)),
        ],
        out_specs=pl.BlockSpec((_BLK, D), lambda i: (i, 0)),
        out_shape=jax.ShapeDtypeStruct((N, D), jnp.float32),
    )(p, w, b.reshape(1, D), x)


@functools.partial(
    pl.kernel,
    mesh=plsc.VectorSubcoreMesh(core_axis_name="c", subcore_axis_name="s"),
    out_type=jax.ShapeDtypeStruct((2, NPAD, H), jnp.float32),
    compiler_params=pltpu.CompilerParams(use_tc_tiling_on_sc=False),
    scratch_types=[
        pltpu.VMEM((CHUNKS, CHUNK), jnp.int32),
        pltpu.VMEM((CHUNKS, CHUNK), jnp.int32),
        pltpu.VMEM((CHUNK, H), jnp.float32),
        pltpu.VMEM((CHUNK, H), jnp.float32),
        pltpu.VMEM((WB, H), jnp.float32),
        pltpu.VMEM_SHARED((NPAD, H), jnp.float32),
        pltpu.SemaphoreType.DMA,
        pltpu.SemaphoreType.DMA,
    ],
)
def _sc_agg(h_hbm, src_hbm, dst_hbm, zeros_hbm, out_hbm,
            src_v, dst_v, buf0, buf1, zbuf, agg_sh, sem0, sem1):
    c = lax.axis_index("c")
    s = lax.axis_index("s")

    pltpu.sync_copy(src_hbm.at[c, s], src_v)
    pltpu.sync_copy(dst_hbm.at[s], dst_v)

    pltpu.sync_copy(zeros_hbm, zbuf)
    base = s * ROWS_PER_TILE

    for k in range(WB_ITERS):
        pltpu.async_copy(zbuf, agg_sh.at[pl.ds(base + k * WB, WB)], sem0)
    for k in range(WB_ITERS):
        pltpu.make_async_copy(zbuf, agg_sh.at[pl.ds(base + k * WB, WB)],
                              sem0).wait()
    plsc.subcore_barrier()

    pltpu.async_copy(h_hbm.at[src_v.at[0]], buf0, sem0)

    def _step(i, _):
        j = i * 2
        pltpu.async_copy(h_hbm.at[src_v.at[j + 1]], buf1, sem1)
        pltpu.make_async_copy(h_hbm.at[src_v.at[j]], buf0, sem0).wait()
        pltpu.sync_copy(buf0, agg_sh.at[dst_v.at[j]], add=True)
        pltpu.async_copy(h_hbm.at[src_v.at[j + 2]], buf0, sem0)
        pltpu.make_async_copy(h_hbm.at[src_v.at[j + 1]], buf1, sem1).wait()
        pltpu.sync_copy(buf1, agg_sh.at[dst_v.at[j + 1]], add=True)
        return 0

    lax.fori_loop(0, (CHUNKS - 2) // 2, _step, 0)
    pltpu.async_copy(h_hbm.at[src_v.at[CHUNKS - 1]], buf1, sem1)
    pltpu.make_async_copy(h_hbm.at[src_v.at[CHUNKS - 2]], buf0, sem0).wait()
    pltpu.sync_copy(buf0, agg_sh.at[dst_v.at[CHUNKS - 2]], add=True)
    pltpu.make_async_copy(h_hbm.at[src_v.at[CHUNKS - 1]], buf1, sem1).wait()
    pltpu.sync_copy(buf1, agg_sh.at[dst_v.at[CHUNKS - 1]], add=True)

    plsc.subcore_barrier()

    wbufs = (buf0, buf1)
    wsems = (sem0, sem1)
    for k in range(WB_ITERS):
        b, sm = wbufs[k % 2], wsems[k % 2]
        if k >= 2:
            pltpu.make_async_copy(
                b, out_hbm.at[c, pl.ds(base + (k - 2) * WB, WB)], sm).wait()
        pltpu.async_copy(agg_sh.at[pl.ds(base + k * WB, WB)], b, sm)
        pltpu.make_async_copy(
            agg_sh.at[pl.ds(base + k * WB, WB)], b, sm).wait()
        pltpu.async_copy(b, out_hbm.at[c, pl.ds(base + k * WB, WB)], sm)
    for k in range(WB_ITERS - 2, WB_ITERS):
        b, sm = wbufs[k % 2], wsems[k % 2]
        pltpu.make_async_copy(
            b, out_hbm.at[c, pl.ds(base + k * WB, WB)], sm).wait()


def kernel(x, edge_index, ln_gamma, ln_beta, W1, b1, W2, b2):
    src = edge_index[0]
    dst = edge_index[1]

    pad = EPAD - E
    src_p = jnp.concatenate([src, jnp.zeros((pad,), jnp.int32)])
    dst_p = jnp.concatenate([dst, jnp.full((pad,), NPAD - 1, jnp.int32)])
    src3 = src_p.reshape(16, CHUNKS, CHUNK)
    src4 = jnp.stack([src3, src3 + N])
    dst3 = dst_p.reshape(16, CHUNKS, CHUNK)
    zeros_tile = jnp.zeros((WB, H), jnp.float32)

    h1 = _tc_pre(x, ln_gamma, ln_beta)
    p1 = _sc_agg(h1, src4, dst3, zeros_tile)
    h2 = _tc_mid(p1, W1, b1)
    p2 = _sc_agg(h2, src4, dst3, zeros_tile)
    return _tc_fin(p2, W2, b2, x)
